# baseline (speedup 1.0000x reference)
import functools

import jax
import jax.numpy as jnp
from jax import lax
from jax.experimental import pallas as pl
from jax.experimental.pallas import tpu as pltpu
from jax.experimental.pallas import tpu_sc as plsc


def _sc_gather(emb_table, idx):
    V, D = emb_table.shape
    B = idx.shape[0]
    info = plsc.get_sparse_core_info()
    NC, NS = info.num_cores, info.num_subcores
    NW = NC * NS
    b_per_w = B // NW
    mesh = plsc.VectorSubcoreMesh(core_axis_name="c", subcore_axis_name="s")

    @functools.partial(
        pl.kernel,
        mesh=mesh,
        out_type=jax.ShapeDtypeStruct((B, D), jnp.float32),
        scratch_types=[
            pltpu.VMEM((b_per_w,), jnp.int32),
            pltpu.VMEM((b_per_w, D), jnp.float32),
            pltpu.SemaphoreType.DMA,
        ],
        compiler_params=pltpu.CompilerParams(use_tc_tiling_on_sc=False),
    )
    def gather_kernel(table_hbm, idx_hbm, out_hbm, idx_v, rows_v, sem):
        wid = lax.axis_index("s") * NC + lax.axis_index("c")
        base = wid * b_per_w
        pltpu.sync_copy(idx_hbm.at[pl.ds(base, b_per_w)], idx_v)
        pltpu.async_copy(table_hbm.at[idx_v], rows_v, sem).wait()
        pltpu.sync_copy(rows_v, out_hbm.at[pl.ds(base, b_per_w)])

    return gather_kernel(emb_table, idx)


def _tc_project_t(x, W_out, b_row):
    B, D = x.shape
    V = W_out.shape[0]
    TV = 2048
    grid = pl.cdiv(V, TV)

    def body(w_ref, x_ref, b_ref, o_ref):
        bias = jnp.transpose(b_ref[...])
        o_ref[...] = lax.dot_general(
            w_ref[...], x_ref[...],
            dimension_numbers=(((1,), (1,)), ((), ())),
            preferred_element_type=jnp.float32,
        ) + bias

    return pl.pallas_call(
        body,
        grid=(grid,),
        in_specs=[
            pl.BlockSpec((TV, D), lambda i: (i, 0)),
            pl.BlockSpec((B, D), lambda i: (0, 0)),
            pl.BlockSpec((1, TV), lambda i: (0, i)),
        ],
        out_specs=pl.BlockSpec((TV, B), lambda i: (i, 0)),
        out_shape=jax.ShapeDtypeStruct((V, B), jnp.float32),
        compiler_params=pltpu.CompilerParams(
            dimension_semantics=("arbitrary",),
        ),
    )(W_out, x, b_row)


_TV = 4096
_Q = 4
_V_TOTAL = 100000
_N_STEPS = 25
_LAST_ROWS = _V_TOTAL - (_N_STEPS - 1) * _TV
_RV = _TV // _Q
_RV_LAST = _LAST_ROWS // _Q


def _fused_body(idx_ref, table_ref, w_ref, b_ref, o_ref, x_vmem, acc, sem):
    i = pl.program_id(0)
    B = x_vmem.shape[0]
    n = pl.num_programs(0)
    slot = lax.rem(i, 3)
    base = slot * _TV

    @pl.when(i == 0)
    def _gather():
        def issue(t, carry):
            pltpu.make_async_copy(
                table_ref.at[pl.ds(idx_ref[t], 1)],
                x_vmem.at[pl.ds(t, 1)],
                sem.at[3, 0],
            ).start()
            return carry
        lax.fori_loop(0, B, issue, 0)
        pltpu.make_async_copy(table_ref.at[pl.ds(0, B)], x_vmem, sem.at[3, 0]).wait()

    @pl.when(i >= 3)
    def _wait_prev():
        for q in range(_Q):
            pltpu.make_async_copy(
                acc.at[pl.ds(base + q * _RV, _RV)],
                o_ref.at[pl.ds((i - 3) * _TV + q * _RV, _RV)],
                sem.at[slot, q],
            ).wait()

    bias = jnp.transpose(b_ref[...])
    acc[pl.ds(base, _TV), :] = lax.dot_general(
        w_ref[...], x_vmem[...],
        dimension_numbers=(((1,), (1,)), ((), ())),
        preferred_element_type=jnp.float32,
    ) + bias

    @pl.when(i < n - 1)
    def _issue_full():
        for q in range(_Q):
            pltpu.make_async_copy(
                acc.at[pl.ds(base + q * _RV, _RV)],
                o_ref.at[pl.ds(i * _TV + q * _RV, _RV)],
                sem.at[slot, q],
            ).start()

    @pl.when(i == n - 1)
    def _last_step():
        for q in range(_Q):
            pltpu.make_async_copy(
                acc.at[pl.ds(base + q * _RV_LAST, _RV_LAST)],
                o_ref.at[pl.ds(i * _TV + q * _RV_LAST, _RV_LAST)],
                sem.at[slot, q],
            ).start()
        for q in range(_Q):
            pltpu.make_async_copy(
                acc.at[pl.ds(base + q * _RV_LAST, _RV_LAST)],
                o_ref.at[pl.ds(q * _RV_LAST, _RV_LAST)],
                sem.at[slot, q],
            ).wait()
        for k in range(1, 3):
            other = lax.rem(slot + 3 - k, 3)
            for q in range(_Q):
                pltpu.make_async_copy(
                    acc.at[pl.ds(other * _TV + q * _RV, _RV)],
                    o_ref.at[pl.ds(q * _RV, _RV)],
                    sem.at[other, q],
                ).wait()


def _fused(center_tokens, emb_table, W_out, b_out):
    idx = center_tokens.astype(jnp.int32)
    V, D = W_out.shape
    B = idx.shape[0]
    logits_t = pl.pallas_call(
        _fused_body,
        grid_spec=pltpu.PrefetchScalarGridSpec(
            num_scalar_prefetch=1,
            grid=(_N_STEPS,),
            in_specs=[
                pl.BlockSpec(memory_space=pltpu.MemorySpace.HBM),
                pl.BlockSpec((_TV, D), lambda i, s: (i, 0)),
                pl.BlockSpec((1, _TV), lambda i, s: (0, i)),
            ],
            out_specs=pl.BlockSpec(memory_space=pltpu.MemorySpace.HBM),
            scratch_shapes=[
                pltpu.VMEM((B, D), jnp.float32),
                pltpu.VMEM((3 * _TV, B), jnp.float32),
                pltpu.SemaphoreType.DMA((4, _Q)),
            ],
        ),
        out_shape=jax.ShapeDtypeStruct((V, B), jnp.float32),
        compiler_params=pltpu.CompilerParams(
            dimension_semantics=("arbitrary",),
        ),
    )(idx, emb_table, W_out, b_out.reshape(1, -1))
    return logits_t.T


def kernel(center_tokens, emb_table, W_out, b_out):
    return _fused(center_tokens, emb_table, W_out, b_out)


# speedup vs baseline: 1.0052x; 1.0052x over previous
import jax
import jax.numpy as jnp
from jax import lax
from jax.experimental import pallas as pl
from jax.experimental.pallas import tpu as pltpu

_TV = 4096


def _fused_body(idx_ref, table_ref, w_ref, b_ref, o_ref, x_vmem, sem):
    i = pl.program_id(0)
    B = x_vmem.shape[0]

    @pl.when(i == 0)
    def _gather():
        def issue(t, carry):
            pltpu.make_async_copy(
                table_ref.at[pl.ds(idx_ref[t], 1)],
                x_vmem.at[pl.ds(t, 1)],
                sem,
            ).start()
            return carry
        lax.fori_loop(0, B, issue, 0)
        pltpu.make_async_copy(table_ref.at[pl.ds(0, B)], x_vmem, sem).wait()

    bias = jnp.transpose(b_ref[...])
    o_ref[...] = lax.dot_general(
        w_ref[...], x_vmem[...],
        dimension_numbers=(((1,), (1,)), ((), ())),
        preferred_element_type=jnp.float32,
    ) + bias


def kernel(center_tokens, emb_table, W_out, b_out):
    idx = center_tokens.astype(jnp.int32)
    V, D = W_out.shape
    B = idx.shape[0]
    grid = pl.cdiv(V, _TV)
    logits_t = pl.pallas_call(
        _fused_body,
        grid_spec=pltpu.PrefetchScalarGridSpec(
            num_scalar_prefetch=1,
            grid=(grid,),
            in_specs=[
                pl.BlockSpec(memory_space=pltpu.MemorySpace.HBM),
                pl.BlockSpec((_TV, D), lambda i, s: (i, 0)),
                pl.BlockSpec((1, _TV), lambda i, s: (0, i)),
            ],
            out_specs=pl.BlockSpec((_TV, B), lambda i, s: (i, 0)),
            scratch_shapes=[
                pltpu.VMEM((B, D), jnp.float32),
                pltpu.SemaphoreType.DMA,
            ],
        ),
        out_shape=jax.ShapeDtypeStruct((V, B), jnp.float32),
        compiler_params=pltpu.CompilerParams(
            dimension_semantics=("arbitrary",),
        ),
    )(idx, emb_table, W_out, b_out.reshape(1, -1))
    return logits_t.T
